# baseline (speedup 1.0000x reference)
import functools

import jax
import jax.numpy as jnp
from jax import lax
from jax.experimental import pallas as pl
from jax.experimental.pallas import tpu as pltpu
from jax.experimental.pallas import tpu_sc as plsc

B = 32
C = 256
P = 4096
S = 64
L = 16
MINPIX = 16.0

C_SC = 192
C_TC = C - C_SC
CHUNK_C = 8
NCHUNK = C_SC // CHUNK_C
TC_BLK = 64


def _sc_body(seg_hbm, fm_hbm, out_hbm, cnt_hbm,
             ids_v, acc_v, cntf_v, cnti_v, scale_v, bufa, bufb,
             sema, semb):
    nc = 2
    wid = lax.axis_index("s") * nc + lax.axis_index("c")
    b = wid

    pltpu.sync_copy(seg_hbm.at[b], ids_v)

    zeros = jnp.zeros((L,), jnp.float32)
    ones = jnp.ones((L,), jnp.float32)

    @plsc.parallel_loop(0, (C_SC * S) // L, unroll=8)
    def _(i):
        acc_v[pl.ds(i * L, L)] = zeros

    for i in range(S // L):
        cntf_v[pl.ds(i * L, L)] = zeros

    @plsc.parallel_loop(0, P // L, unroll=4)
    def _(g):
        ids = ids_v[pl.ds(g * L, L)]
        plsc.addupdate_scatter(cntf_v, [ids], ones)

    def start(chunk, buf, sem):
        pltpu.make_async_copy(
            fm_hbm.at[b, pl.ds(C_TC + chunk * CHUNK_C, CHUNK_C), :],
            buf, sem).start()

    def wait(buf, sem):
        pltpu.make_async_copy(
            fm_hbm.at[b, pl.ds(0, CHUNK_C), :], buf, sem).wait()

    start(0, bufa, sema)

    def compute(chunk, buf):
        c0 = chunk * CHUNK_C

        @plsc.parallel_loop(0, P // L, unroll=4)
        def _(g):
            base = ids_v[pl.ds(g * L, L)] + c0 * S
            off = g * L
            for cc in range(CHUNK_C):
                vals = buf[cc, pl.ds(off, L)]
                plsc.addupdate_scatter(acc_v, [base + cc * S], vals)

    def mbody(k, _):
        start(2 * k + 1, bufb, semb)
        wait(bufa, sema)
        compute(2 * k, bufa)

        @pl.when(k < NCHUNK // 2 - 1)
        def _():
            start(2 * k + 2, bufa, sema)

        wait(bufb, semb)
        compute(2 * k + 1, bufb)
        return 0

    lax.fori_loop(0, NCHUNK // 2, mbody, 0)

    for i in range(S // L):
        cnt = cntf_v[pl.ds(i * L, L)]
        sc = jnp.where(cnt >= MINPIX, 1.0 / jnp.maximum(cnt, 1.0), 0.0)
        scale_v[pl.ds(i * L, L)] = sc
        cnti_v[pl.ds(i * L, L)] = cnt.astype(jnp.int32)

    @plsc.parallel_loop(0, C_SC, unroll=2)
    def _(c):
        for j in range(S // L):
            sv = scale_v[pl.ds(j * L, L)]
            o = c * S + j * L
            acc_v[pl.ds(o, L)] = acc_v[pl.ds(o, L)] * sv

    pltpu.sync_copy(acc_v, out_hbm.at[b])
    pltpu.sync_copy(cnti_v, cnt_hbm.at[b])


@jax.jit
def _sc_call(seg, fm):
    mesh = plsc.VectorSubcoreMesh(core_axis_name="c", subcore_axis_name="s")
    f = functools.partial(
        pl.kernel,
        mesh=mesh,
        compiler_params=pltpu.CompilerParams(needs_layout_passes=False),
        out_type=[
            jax.ShapeDtypeStruct((B, C_SC * S), jnp.float32),
            jax.ShapeDtypeStruct((B, S), jnp.int32),
        ],
        scratch_types=[
            pltpu.VMEM((P,), jnp.int32),
            pltpu.VMEM((C_SC * S,), jnp.float32),
            pltpu.VMEM((S,), jnp.float32),
            pltpu.VMEM((S,), jnp.int32),
            pltpu.VMEM((S,), jnp.float32),
            pltpu.VMEM((CHUNK_C, P), jnp.float32),
            pltpu.VMEM((CHUNK_C, P), jnp.float32),
            pltpu.SemaphoreType.DMA,
            pltpu.SemaphoreType.DMA,
        ],
    )(_sc_body)
    return f(seg, fm)


def _tc_body(seg_ref, fm_ref, out_ref):
    seg = seg_ref[0, 0, :]
    onehot = (seg[:, None] == jax.lax.broadcasted_iota(jnp.int32, (1, S), 1)
              ).astype(jnp.float32)
    counts = jnp.sum(onehot, axis=0)
    scale = jnp.where(counts >= MINPIX, 1.0 / jnp.maximum(counts, 1.0), 0.0)
    sums = jnp.dot(fm_ref[0], onehot, preferred_element_type=jnp.float32)
    out_ref[0, 0] = jnp.transpose(sums * scale[None, :])


@jax.jit
def _tc_call(seg3, fm):
    return pl.pallas_call(
        _tc_body,
        grid=(B, C_TC // TC_BLK),
        in_specs=[
            pl.BlockSpec((1, 1, P), lambda b, cb: (b, 0, 0)),
            pl.BlockSpec((1, TC_BLK, P), lambda b, cb: (b, cb, 0)),
        ],
        out_specs=pl.BlockSpec((1, 1, S, TC_BLK), lambda b, cb: (b, cb, 0, 0)),
        out_shape=jax.ShapeDtypeStruct(
            (B, C_TC // TC_BLK, S, TC_BLK), jnp.float32),
        compiler_params=pltpu.CompilerParams(
            dimension_semantics=("parallel", "arbitrary")),
    )(seg3, fm)


def kernel(segment_ids, sam_encoder_embeddings):
    fm3 = jnp.squeeze(sam_encoder_embeddings, axis=1).reshape(B, C, P)
    seg = segment_ids.reshape(B, P)
    seg3 = segment_ids.reshape(B, 1, P)

    out_sc, cnt = _sc_call(seg, fm3)
    out_tc = _tc_call(seg3, fm3)

    out_sc = out_sc.reshape(B, C_SC, S).transpose(0, 2, 1)
    out_tc = out_tc.transpose(0, 2, 1, 3).reshape(B, S, C_TC)
    out = jnp.concatenate([out_tc, out_sc], axis=2)
    valid = cnt >= int(MINPIX)
    return out, valid


# speedup vs baseline: 1.1716x; 1.1716x over previous
import functools

import jax
import jax.numpy as jnp
from jax import lax
from jax.experimental import pallas as pl
from jax.experimental.pallas import tpu as pltpu
from jax.experimental.pallas import tpu_sc as plsc

B = 32
C = 256
P = 4096
S = 64
L = 16
MINPIX = 16.0

C_SC = 128
C_TC = C - C_SC
CHUNK_C = 8
NCHUNK = C_SC // CHUNK_C
TC_BLK = 128


def _sc_body(seg_hbm, fm_hbm, out_hbm, cnt_hbm,
             ids_v, acc_v, cntf_v, cnti_v, scale_v, bufa, bufb,
             sema, semb):
    nc = 2
    wid = lax.axis_index("s") * nc + lax.axis_index("c")
    b = wid

    pltpu.sync_copy(seg_hbm.at[b], ids_v)

    zeros = jnp.zeros((L,), jnp.float32)
    ones = jnp.ones((L,), jnp.float32)

    @plsc.parallel_loop(0, (C_SC * S) // L, unroll=8)
    def _(i):
        acc_v[pl.ds(i * L, L)] = zeros

    for i in range(S // L):
        cntf_v[pl.ds(i * L, L)] = zeros

    @plsc.parallel_loop(0, P // L, unroll=4)
    def _(g):
        ids = ids_v[pl.ds(g * L, L)]
        plsc.addupdate_scatter(cntf_v, [ids], ones)

    def start(chunk, buf, sem):
        pltpu.make_async_copy(
            fm_hbm.at[b, pl.ds(C_TC + chunk * CHUNK_C, CHUNK_C), :],
            buf, sem).start()

    def wait(buf, sem):
        pltpu.make_async_copy(
            fm_hbm.at[b, pl.ds(0, CHUNK_C), :], buf, sem).wait()

    start(0, bufa, sema)

    def compute(chunk, buf):
        c0 = chunk * CHUNK_C

        @plsc.parallel_loop(0, P // L, unroll=4)
        def _(g):
            base = ids_v[pl.ds(g * L, L)] + c0 * S
            off = g * L
            for cc in range(CHUNK_C):
                vals = buf[cc, pl.ds(off, L)]
                plsc.addupdate_scatter(acc_v, [base + cc * S], vals)

    def mbody(k, _):
        start(2 * k + 1, bufb, semb)
        wait(bufa, sema)
        compute(2 * k, bufa)

        @pl.when(k < NCHUNK // 2 - 1)
        def _():
            start(2 * k + 2, bufa, sema)

        wait(bufb, semb)
        compute(2 * k + 1, bufb)
        return 0

    lax.fori_loop(0, NCHUNK // 2, mbody, 0)

    for i in range(S // L):
        cnt = cntf_v[pl.ds(i * L, L)]
        sc = jnp.where(cnt >= MINPIX, 1.0 / jnp.maximum(cnt, 1.0), 0.0)
        scale_v[pl.ds(i * L, L)] = sc
        cnti_v[pl.ds(i * L, L)] = cnt.astype(jnp.int32)

    @plsc.parallel_loop(0, C_SC, unroll=2)
    def _(c):
        for j in range(S // L):
            sv = scale_v[pl.ds(j * L, L)]
            o = c * S + j * L
            acc_v[pl.ds(o, L)] = acc_v[pl.ds(o, L)] * sv

    pltpu.sync_copy(acc_v, out_hbm.at[b])
    pltpu.sync_copy(cnti_v, cnt_hbm.at[b])


@jax.jit
def _sc_call(seg, fm):
    mesh = plsc.VectorSubcoreMesh(core_axis_name="c", subcore_axis_name="s")
    f = functools.partial(
        pl.kernel,
        mesh=mesh,
        compiler_params=pltpu.CompilerParams(needs_layout_passes=False),
        out_type=[
            jax.ShapeDtypeStruct((B, C_SC * S), jnp.float32),
            jax.ShapeDtypeStruct((B, S), jnp.int32),
        ],
        scratch_types=[
            pltpu.VMEM((P,), jnp.int32),
            pltpu.VMEM((C_SC * S,), jnp.float32),
            pltpu.VMEM((S,), jnp.float32),
            pltpu.VMEM((S,), jnp.int32),
            pltpu.VMEM((S,), jnp.float32),
            pltpu.VMEM((CHUNK_C, P), jnp.float32),
            pltpu.VMEM((CHUNK_C, P), jnp.float32),
            pltpu.SemaphoreType.DMA,
            pltpu.SemaphoreType.DMA,
        ],
    )(_sc_body)
    return f(seg, fm)


def _tc_body(seg_ref, fm_ref, out_ref):
    seg = seg_ref[0, 0, :]
    onehot = (seg[:, None] == jax.lax.broadcasted_iota(jnp.int32, (1, S), 1)
              ).astype(jnp.float32)
    counts = jnp.sum(onehot, axis=0)
    scale = jnp.where(counts >= MINPIX, 1.0 / jnp.maximum(counts, 1.0), 0.0)
    sums = jnp.dot(fm_ref[0], onehot, preferred_element_type=jnp.float32)
    out_ref[0, 0] = jnp.transpose(sums * scale[None, :])


@jax.jit
def _tc_call(seg3, fm):
    return pl.pallas_call(
        _tc_body,
        grid=(B, C_TC // TC_BLK),
        in_specs=[
            pl.BlockSpec((1, 1, P), lambda b, cb: (b, 0, 0)),
            pl.BlockSpec((1, TC_BLK, P), lambda b, cb: (b, cb, 0)),
        ],
        out_specs=pl.BlockSpec((1, 1, S, TC_BLK), lambda b, cb: (b, cb, 0, 0)),
        out_shape=jax.ShapeDtypeStruct(
            (B, C_TC // TC_BLK, S, TC_BLK), jnp.float32),
        compiler_params=pltpu.CompilerParams(
            dimension_semantics=("parallel", "arbitrary")),
    )(seg3, fm)


def kernel(segment_ids, sam_encoder_embeddings):
    fm3 = jnp.squeeze(sam_encoder_embeddings, axis=1).reshape(B, C, P)
    seg = segment_ids.reshape(B, P)
    seg3 = segment_ids.reshape(B, 1, P)

    out_sc, cnt = _sc_call(seg, fm3)
    out_tc = _tc_call(seg3, fm3)

    out_sc = out_sc.reshape(B, C_SC, S).transpose(0, 2, 1)
    out_tc = out_tc.transpose(0, 2, 1, 3).reshape(B, S, C_TC)
    out = jnp.concatenate([out_tc, out_sc], axis=2)
    valid = cnt >= int(MINPIX)
    return out, valid
